# Initial kernel scaffold; baseline (speedup 1.0000x reference)
import jax
import jax.numpy as jnp
from jax.experimental import pallas as pl


def kernel(x, training, class_logits, feat0, feat_self, feat_pair, struct):
    raise NotImplementedError("write your pallas kernel here")


# speedup vs baseline: 3.8388x; 3.8388x over previous
import functools

import jax
import jax.numpy as jnp
from jax import lax
from jax.experimental import pallas as pl
from jax.experimental.pallas import tpu as pltpu
from jax.experimental.pallas import tpu_sc as plsc

N_FEATURES = 26
N_UNIQUE = 100
N_CLASSES = 64
BATCH = 16384

NC = 2
NS = 16
NW = NC * NS
SAMP_PER_W = BATCH // NW
CHUNK = 128
NCHUNKS = SAMP_PER_W // CHUNK
PAIR_ROWS = N_UNIQUE * N_UNIQUE


def _pair_lse_body(fp_ref, out_ref):
    xb = fp_ref[0]
    m = jnp.max(xb, axis=0, keepdims=True)
    s = jnp.sum(jnp.exp(xb - m), axis=0, keepdims=True)
    out_ref[0] = -(jnp.log(s) + m)


def _pair_lse(feat_pair):
    fp = feat_pair.reshape(N_FEATURES - 1, N_UNIQUE, N_UNIQUE * N_CLASSES)
    out = pl.pallas_call(
        _pair_lse_body,
        grid=(N_FEATURES - 1,),
        in_specs=[pl.BlockSpec((1, N_UNIQUE, N_UNIQUE * N_CLASSES),
                               lambda i: (i, 0, 0))],
        out_specs=pl.BlockSpec((1, 1, N_UNIQUE * N_CLASSES), lambda i: (i, 0, 0)),
        out_shape=jax.ShapeDtypeStruct(
            (N_FEATURES - 1, 1, N_UNIQUE * N_CLASSES), jnp.float32),
    )(fp)
    return out.reshape(N_FEATURES - 1, N_UNIQUE, N_CLASSES)


def _small_prep_body(cl_ref, f0_ref, fs_ref, st_ref, nlse_ref, ss_ref, kf_ref):
    cl = cl_ref[...]
    m = jnp.max(cl, axis=1, keepdims=True)
    base = cl - (jnp.log(jnp.sum(jnp.exp(cl - m), axis=1, keepdims=True)) + m)

    f0 = f0_ref[...]
    m0 = jnp.max(f0, axis=0, keepdims=True)
    lse0 = jnp.log(jnp.sum(jnp.exp(f0 - m0), axis=0, keepdims=True)) + m0
    ss_ref[0] = f0 - lse0 + base

    fs = fs_ref[...]
    m1 = jnp.max(fs, axis=1, keepdims=True)
    lse1 = jnp.log(jnp.sum(jnp.exp(fs - m1), axis=1, keepdims=True)) + m1
    fsn = fs - lse1

    st = st_ref[...]
    k2 = st[:, 1:2] > st[:, 0:1]
    kb = k2[:, :, None]
    ss_ref[1:N_FEATURES] = jnp.where(kb, nlse_ref[...], fsn)

    kf = jnp.concatenate(
        [jnp.zeros((1, 16), jnp.int32),
         jnp.broadcast_to(k2.astype(jnp.int32), (N_FEATURES - 1, 16))], axis=0)
    kf_ref[...] = kf


def _small_prep(class_logits, feat0, feat_self, struct, nlse):
    return pl.pallas_call(
        _small_prep_body,
        out_shape=(
            jax.ShapeDtypeStruct((N_FEATURES, N_UNIQUE, N_CLASSES), jnp.float32),
            jax.ShapeDtypeStruct((N_FEATURES, 16), jnp.int32),
        ),
    )(class_logits.reshape(1, N_CLASSES), feat0, feat_self, struct, nlse)


def _accum(acc, buf):
    def body(r, carry):
        for k4 in range(N_CLASSES // 16):
            sl = pl.ds(k4 * 16, 16)
            plsc.addupdate(acc.at[r, sl], buf[r, sl])
        return carry
    lax.fori_loop(0, CHUNK, body, 0, unroll=4)


@functools.cache
def _build_sc_gather():
  mesh = plsc.VectorSubcoreMesh(
      core_axis_name="c", subcore_axis_name="s", num_cores=NC, num_subcores=NS)

  @functools.partial(
      pl.kernel,
      out_type=jax.ShapeDtypeStruct((BATCH, N_CLASSES), jnp.float32),
      mesh=mesh,
      scratch_types=[
        pltpu.VMEM((N_FEATURES, CHUNK), jnp.int32),
        pltpu.VMEM((N_FEATURES, 16), jnp.int32),
        pltpu.VMEM((CHUNK,), jnp.int32),
        pltpu.VMEM((CHUNK,), jnp.int32),
        pltpu.VMEM((CHUNK, N_CLASSES), jnp.float32),
        pltpu.VMEM((CHUNK, N_CLASSES), jnp.float32),
          pltpu.VMEM((CHUNK, N_CLASSES), jnp.float32),
          pltpu.SemaphoreType.DMA,
          pltpu.SemaphoreType.DMA,
      ],
      compiler_params=pltpu.CompilerParams(use_tc_tiling_on_sc=False),
  )
  def _sc_gather(xT_hbm, ss_hbm, pair_hbm, kf_hbm, out_hbm,
                 xv, kv, idxs, idxp, buf_s, buf_p, acc, sem_s, sem_p):
      wid = lax.axis_index("s") * NC + lax.axis_index("c")
      pltpu.sync_copy(kf_hbm, kv)

      def chunk_body(ci, carry):
          cbase = wid * SAMP_PER_W + ci * CHUNK
          pltpu.sync_copy(xT_hbm.at[:, pl.ds(cbase, CHUNK)], xv)

          for c8 in range(CHUNK // 16):
              sl = pl.ds(c8 * 16, 16)
              idxs[sl] = xv[0, sl]
          pltpu.async_copy(ss_hbm.at[idxs], acc, sem_s).wait()

          def feat_body(i, fcarry):
              kvec = kv[i, :]
              im1 = i - 1
              for c8 in range(CHUNK // 16):
                  sl = pl.ds(c8 * 16, 16)
                  xc = xv[i, sl]
                  xp = xv[im1, sl]
                  idxs[sl] = jnp.where(kvec > 0, xp, xc) + i * N_UNIQUE
                  idxp[sl] = im1 * PAIR_ROWS + xc * N_UNIQUE + xp
              pltpu.async_copy(ss_hbm.at[idxs], buf_s, sem_s).wait()
              _accum(acc, buf_s)

              ks = kvec[0]
              @pl.when(ks > 0)
              def _():
                  pltpu.async_copy(pair_hbm.at[idxp], buf_p, sem_p).wait()
                  _accum(acc, buf_p)
              return fcarry

          lax.fori_loop(1, N_FEATURES, feat_body, 0)
          pltpu.sync_copy(acc, out_hbm.at[pl.ds(cbase, CHUNK), :])
          return carry

      lax.fori_loop(0, NCHUNKS, chunk_body, 0)

  return _sc_gather


def kernel(x, training, class_logits, feat0, feat_self, feat_pair, struct):
    del training
    xT = x.astype(jnp.int32).T
    pair_flat = feat_pair.reshape(-1, N_CLASSES)
    nlse = _pair_lse(feat_pair)
    ss, kf = _small_prep(class_logits, feat0, feat_self, struct, nlse)
    ss_flat = ss.reshape(-1, N_CLASSES)
    return _build_sc_gather()(xT, ss_flat, pair_flat, kf)


# speedup vs baseline: 4.8771x; 1.2705x over previous
import functools

import jax
import jax.numpy as jnp
from jax import lax
from jax.experimental import pallas as pl
from jax.experimental.pallas import tpu as pltpu
from jax.experimental.pallas import tpu_sc as plsc

N_FEATURES = 26
N_UNIQUE = 100
N_CLASSES = 64
BATCH = 16384

NC = 2
NS = 16
NW = NC * NS
SAMP_PER_W = BATCH // NW
CHUNK = 128
NCHUNKS = SAMP_PER_W // CHUNK
PAIR_ROWS = N_UNIQUE * N_UNIQUE


def _pair_lse_body(fpT_ref, out_ref):
    xb = fpT_ref[0]
    m = jnp.max(xb, axis=0)
    s = jnp.sum(jnp.exp(xb - m[None]), axis=0)
    out_ref[0] = -(jnp.log(s) + m)


def _pair_lse(feat_pair):
    fpT = jnp.transpose(feat_pair, (0, 1, 3, 2))
    out = pl.pallas_call(
        _pair_lse_body,
        grid=(N_FEATURES - 1,),
        in_specs=[pl.BlockSpec((1, N_UNIQUE, N_CLASSES, N_UNIQUE),
                               lambda i: (i, 0, 0, 0))],
        out_specs=pl.BlockSpec((1, N_CLASSES, N_UNIQUE), lambda i: (i, 0, 0)),
        out_shape=jax.ShapeDtypeStruct(
            (N_FEATURES - 1, N_CLASSES, N_UNIQUE), jnp.float32),
    )(fpT)
    return out


def _small_prep_body(cl_ref, f0_ref, fs_ref, st_ref, nlse_ref, ss_ref, kf_ref):
    cl = cl_ref[...]
    m = jnp.max(cl, axis=1, keepdims=True)
    base = cl - (jnp.log(jnp.sum(jnp.exp(cl - m), axis=1, keepdims=True)) + m)

    f0 = f0_ref[...]
    m0 = jnp.max(f0, axis=0, keepdims=True)
    lse0 = jnp.log(jnp.sum(jnp.exp(f0 - m0), axis=0, keepdims=True)) + m0
    ss_ref[0] = f0 - lse0 + base

    fs = fs_ref[...]
    m1 = jnp.max(fs, axis=1, keepdims=True)
    lse1 = jnp.log(jnp.sum(jnp.exp(fs - m1), axis=1, keepdims=True)) + m1
    fsn = fs - lse1

    st = st_ref[...]
    k2 = st[:, 1:2] > st[:, 0:1]
    kb = k2[:, :, None]
    nlse = jnp.swapaxes(nlse_ref[...], 1, 2)
    ss_ref[1:N_FEATURES] = jnp.where(kb, nlse, fsn)

    kf = jnp.concatenate(
        [jnp.zeros((1, 16), jnp.int32),
         jnp.broadcast_to(k2.astype(jnp.int32), (N_FEATURES - 1, 16))], axis=0)
    kf_ref[...] = kf


def _small_prep(class_logits, feat0, feat_self, struct, nlse):
    return pl.pallas_call(
        _small_prep_body,
        out_shape=(
            jax.ShapeDtypeStruct((N_FEATURES, N_UNIQUE, N_CLASSES), jnp.float32),
            jax.ShapeDtypeStruct((N_FEATURES, 16), jnp.int32),
        ),
    )(class_logits.reshape(1, N_CLASSES), feat0, feat_self, struct, nlse)


def _accum(acc, buf):
    def body(r, carry):
        for k4 in range(N_CLASSES // 16):
            sl = pl.ds(k4 * 16, 16)
            plsc.addupdate(acc.at[r, sl], buf[r, sl])
        return carry
    lax.fori_loop(0, CHUNK, body, 0, unroll=4)


@functools.cache
def _build_sc_gather():
  mesh = plsc.VectorSubcoreMesh(
      core_axis_name="c", subcore_axis_name="s", num_cores=NC, num_subcores=NS)

  @functools.partial(
      pl.kernel,
      out_type=jax.ShapeDtypeStruct((BATCH, N_CLASSES), jnp.float32),
      mesh=mesh,
      scratch_types=[
        pltpu.VMEM((N_FEATURES, CHUNK), jnp.int32),
        pltpu.VMEM((N_FEATURES, 16), jnp.int32),
        pltpu.VMEM((CHUNK,), jnp.int32),
        pltpu.VMEM((CHUNK,), jnp.int32),
        pltpu.VMEM((CHUNK, N_CLASSES), jnp.float32),
        pltpu.VMEM((CHUNK, N_CLASSES), jnp.float32),
          pltpu.VMEM((CHUNK, N_CLASSES), jnp.float32),
          pltpu.SemaphoreType.DMA,
          pltpu.SemaphoreType.DMA,
      ],
      compiler_params=pltpu.CompilerParams(use_tc_tiling_on_sc=False),
  )
  def _sc_gather(xT_hbm, ss_hbm, pair_hbm, kf_hbm, out_hbm,
                 xv, kv, idxs, idxp, buf_s, buf_p, acc, sem_s, sem_p):
      wid = lax.axis_index("s") * NC + lax.axis_index("c")
      pltpu.sync_copy(kf_hbm, kv)

      def chunk_body(ci, carry):
          cbase = wid * SAMP_PER_W + ci * CHUNK
          pltpu.sync_copy(xT_hbm.at[:, pl.ds(cbase, CHUNK)], xv)

          for c8 in range(CHUNK // 16):
              sl = pl.ds(c8 * 16, 16)
              idxs[sl] = xv[0, sl]
          pltpu.async_copy(ss_hbm.at[idxs], acc, sem_s).wait()

          def feat_body(i, fcarry):
              kvec = kv[i, :]
              im1 = i - 1
              for c8 in range(CHUNK // 16):
                  sl = pl.ds(c8 * 16, 16)
                  xc = xv[i, sl]
                  xp = xv[im1, sl]
                  idxs[sl] = jnp.where(kvec > 0, xp, xc) + i * N_UNIQUE
                  idxp[sl] = im1 * PAIR_ROWS + xc * N_UNIQUE + xp
              pltpu.async_copy(ss_hbm.at[idxs], buf_s, sem_s).wait()
              _accum(acc, buf_s)

              ks = kvec[0]
              @pl.when(ks > 0)
              def _():
                  pltpu.async_copy(pair_hbm.at[idxp], buf_p, sem_p).wait()
                  _accum(acc, buf_p)
              return fcarry

          lax.fori_loop(1, N_FEATURES, feat_body, 0)
          pltpu.sync_copy(acc, out_hbm.at[pl.ds(cbase, CHUNK), :])
          return carry

      lax.fori_loop(0, NCHUNKS, chunk_body, 0)

  return _sc_gather


def kernel(x, training, class_logits, feat0, feat_self, feat_pair, struct):
    del training
    xT = x.astype(jnp.int32).T
    pair_flat = feat_pair.reshape(-1, N_CLASSES)
    nlse = _pair_lse(feat_pair)
    ss, kf = _small_prep(class_logits, feat0, feat_self, struct, nlse)
    ss_flat = ss.reshape(-1, N_CLASSES)
    return _build_sc_gather()(xT, ss_flat, pair_flat, kf)


# speedup vs baseline: 6.2786x; 1.2874x over previous
import functools

import jax
import jax.numpy as jnp
from jax import lax
from jax.experimental import pallas as pl
from jax.experimental.pallas import tpu as pltpu
from jax.experimental.pallas import tpu_sc as plsc

N_FEATURES = 26
N_UNIQUE = 100
N_CLASSES = 64
BATCH = 16384

NC = 2
NS = 16
NW = NC * NS
SAMP_PER_W = BATCH // NW
CHUNK = 256
NCHUNKS = SAMP_PER_W // CHUNK
NIDX = CHUNK // 128
PAIR_ROWS = N_UNIQUE * N_UNIQUE


def _pair_lse_body(fpT_ref, out_ref):
    xb = fpT_ref[0]
    m = jnp.max(xb, axis=0)
    s = jnp.sum(jnp.exp(xb - m[None]), axis=0)
    out_ref[0] = -(jnp.log(s) + m)


def _pair_lse(feat_pair):
    fpT = jnp.transpose(feat_pair, (0, 1, 3, 2))
    out = pl.pallas_call(
        _pair_lse_body,
        grid=(N_FEATURES - 1,),
        in_specs=[pl.BlockSpec((1, N_UNIQUE, N_CLASSES, N_UNIQUE),
                               lambda i: (i, 0, 0, 0))],
        out_specs=pl.BlockSpec((1, N_CLASSES, N_UNIQUE), lambda i: (i, 0, 0)),
        out_shape=jax.ShapeDtypeStruct(
            (N_FEATURES - 1, N_CLASSES, N_UNIQUE), jnp.float32),
    )(fpT)
    return out


def _small_prep_body(cl_ref, f0_ref, fs_ref, st_ref, nlse_ref, ss_ref, kf_ref):
    cl = cl_ref[...]
    m = jnp.max(cl, axis=1, keepdims=True)
    base = cl - (jnp.log(jnp.sum(jnp.exp(cl - m), axis=1, keepdims=True)) + m)

    f0 = f0_ref[...]
    m0 = jnp.max(f0, axis=0, keepdims=True)
    lse0 = jnp.log(jnp.sum(jnp.exp(f0 - m0), axis=0, keepdims=True)) + m0
    ss_ref[0] = f0 - lse0 + base

    fs = fs_ref[...]
    m1 = jnp.max(fs, axis=1, keepdims=True)
    lse1 = jnp.log(jnp.sum(jnp.exp(fs - m1), axis=1, keepdims=True)) + m1
    fsn = fs - lse1

    st = st_ref[...]
    k2 = st[:, 1:2] > st[:, 0:1]
    kb = k2[:, :, None]
    nlse = jnp.swapaxes(nlse_ref[...], 1, 2)
    ss_ref[1:N_FEATURES] = jnp.where(kb, nlse, fsn)

    kf = jnp.concatenate(
        [jnp.zeros((1, 16), jnp.int32),
         jnp.broadcast_to(k2.astype(jnp.int32), (N_FEATURES - 1, 16))], axis=0)
    kf_ref[...] = kf


def _small_prep(class_logits, feat0, feat_self, struct, nlse):
    return pl.pallas_call(
        _small_prep_body,
        out_shape=(
            jax.ShapeDtypeStruct((N_FEATURES, N_UNIQUE, N_CLASSES), jnp.float32),
            jax.ShapeDtypeStruct((N_FEATURES, 16), jnp.int32),
        ),
    )(class_logits.reshape(1, N_CLASSES), feat0, feat_self, struct, nlse)


def _accum(acc, buf, slot):
    def body(r, carry):
        for k4 in range(N_CLASSES // 16):
            sl = pl.ds(k4 * 16, 16)
            plsc.addupdate(acc.at[r, sl], buf[slot, r, sl])
        return carry
    lax.fori_loop(0, CHUNK, body, 0, unroll=4)


@functools.cache
def _build_sc_gather():
  mesh = plsc.VectorSubcoreMesh(
      core_axis_name="c", subcore_axis_name="s", num_cores=NC, num_subcores=NS)

  @functools.partial(
      pl.kernel,
      out_type=jax.ShapeDtypeStruct((BATCH, N_CLASSES), jnp.float32),
      mesh=mesh,
      scratch_types=[
          pltpu.VMEM((N_FEATURES, CHUNK), jnp.int32),
          pltpu.VMEM((N_FEATURES, 16), jnp.int32),
          pltpu.VMEM((2, NIDX, 128), jnp.int32),
          pltpu.VMEM((2, NIDX, 128), jnp.int32),
          pltpu.VMEM((2, CHUNK, N_CLASSES), jnp.float32),
          pltpu.VMEM((2, CHUNK, N_CLASSES), jnp.float32),
          pltpu.VMEM((CHUNK, N_CLASSES), jnp.float32),
          pltpu.SemaphoreType.DMA,
          pltpu.SemaphoreType.DMA,
          pltpu.SemaphoreType.DMA,
          pltpu.SemaphoreType.DMA,
      ],
      compiler_params=pltpu.CompilerParams(use_tc_tiling_on_sc=False),
  )
  def _sc_gather(xT_hbm, ss_hbm, pair_hbm, kf_hbm, out_hbm,
                 xv, kv, idxs, idxp, buf_s, buf_p, acc,
                 sem_s0, sem_s1, sem_p0, sem_p1):
      wid = lax.axis_index("s") * NC + lax.axis_index("c")
      pltpu.sync_copy(kf_hbm, kv)
      sem_s = (sem_s0, sem_s1)
      sem_p = (sem_p0, sem_p1)

      def chunk_body(ci, carry):
          cbase = wid * SAMP_PER_W + ci * CHUNK
          pltpu.sync_copy(xT_hbm.at[:, pl.ds(cbase, CHUNK)], xv)

          def compute_idx(i, slot):
              kvec = kv[i, :]
              for c in range(CHUNK // 16):
                  j, o = divmod(c * 16, 128)
                  sl = pl.ds(c * 16, 16)
                  if i == 0:
                      idxs[slot, j, pl.ds(o, 16)] = xv[0, sl]
                  else:
                      xc = xv[i, sl]
                      xp = xv[i - 1, sl]
                      idxs[slot, j, pl.ds(o, 16)] = (
                          jnp.where(kvec > 0, xp, xc) + i * N_UNIQUE)
                      idxp[slot, j, pl.ds(o, 16)] = (
                          (i - 1) * PAIR_ROWS + xc * N_UNIQUE + xp)

          def fire(i, slot):
              dst = acc if i == 0 else buf_s.at[slot]
              hs, hp = [], []
              for j in range(NIDX):
                  hs.append(pltpu.make_async_copy(
                      ss_hbm.at[idxs.at[slot, j]],
                      dst.at[pl.ds(j * 128, 128)], sem_s[slot]))
                  hs[-1].start()
              if i > 0:
                  for j in range(NIDX):
                      hp.append(pltpu.make_async_copy(
                          pair_hbm.at[idxp.at[slot, j]],
                          buf_p.at[slot, pl.ds(j * 128, 128)], sem_p[slot]))
                      pl.when(kv[i, :][0] > 0)(hp[-1].start)
              return hs, hp

          def drain(i, slot, handles):
              hs, hp = handles
              for hj in hs:
                  hj.wait()
              if i > 0:
                  _accum(acc, buf_s, slot)

                  @pl.when(kv[i, :][0] > 0)
                  def _():
                      for hj in hp:
                          hj.wait()
                      _accum(acc, buf_p, slot)

          compute_idx(0, 0)
          h = {0: fire(0, 0)}
          compute_idx(1, 1)
          h[1] = fire(1, 1)
          drain(0, 0, h[0])
          for i in range(2, N_FEATURES):
              s = i & 1
              compute_idx(i, s)
              h[i] = fire(i, s)
              drain(i - 1, 1 - s, h[i - 1])
          drain(N_FEATURES - 1, (N_FEATURES - 1) & 1, h[N_FEATURES - 1])

          pltpu.sync_copy(acc, out_hbm.at[pl.ds(cbase, CHUNK), :])
          return carry

      lax.fori_loop(0, NCHUNKS, chunk_body, 0)

  return _sc_gather


def kernel(x, training, class_logits, feat0, feat_self, feat_pair, struct):
    del training
    xT = x.astype(jnp.int32).T
    pair_flat = feat_pair.reshape(-1, N_CLASSES)
    nlse = _pair_lse(feat_pair)
    ss, kf = _small_prep(class_logits, feat0, feat_self, struct, nlse)
    ss_flat = ss.reshape(-1, N_CLASSES)
    return _build_sc_gather()(xT, ss_flat, pair_flat, kf)


# speedup vs baseline: 6.6600x; 1.0608x over previous
import functools

import jax
import jax.numpy as jnp
from jax import lax
from jax.experimental import pallas as pl
from jax.experimental.pallas import tpu as pltpu
from jax.experimental.pallas import tpu_sc as plsc

N_FEATURES = 26
N_UNIQUE = 100
N_CLASSES = 64
BATCH = 16384

NC = 2
NS = 16
NW = NC * NS
SAMP_PER_W = BATCH // NW
CHUNK = 128
NCHUNKS = SAMP_PER_W // CHUNK
NIDX = CHUNK // 128
NSLOT = 4
PAIR_ROWS = N_UNIQUE * N_UNIQUE


def _pair_lse_body(fpT_ref, out_ref):
    xb = fpT_ref[0]
    m = jnp.max(xb, axis=0)
    s = jnp.sum(jnp.exp(xb - m[None]), axis=0)
    out_ref[0] = -(jnp.log(s) + m)


def _pair_lse(feat_pair):
    fpT = jnp.transpose(feat_pair, (0, 1, 3, 2))
    out = pl.pallas_call(
        _pair_lse_body,
        grid=(N_FEATURES - 1,),
        in_specs=[pl.BlockSpec((1, N_UNIQUE, N_CLASSES, N_UNIQUE),
                               lambda i: (i, 0, 0, 0))],
        out_specs=pl.BlockSpec((1, N_CLASSES, N_UNIQUE), lambda i: (i, 0, 0)),
        out_shape=jax.ShapeDtypeStruct(
            (N_FEATURES - 1, N_CLASSES, N_UNIQUE), jnp.float32),
    )(fpT)
    return out


def _small_prep_body(cl_ref, f0_ref, fs_ref, st_ref, nlse_ref, ss_ref, kf_ref):
    cl = cl_ref[...]
    m = jnp.max(cl, axis=1, keepdims=True)
    base = cl - (jnp.log(jnp.sum(jnp.exp(cl - m), axis=1, keepdims=True)) + m)

    f0 = f0_ref[...]
    m0 = jnp.max(f0, axis=0, keepdims=True)
    lse0 = jnp.log(jnp.sum(jnp.exp(f0 - m0), axis=0, keepdims=True)) + m0
    ss_ref[0] = f0 - lse0 + base

    fs = fs_ref[...]
    m1 = jnp.max(fs, axis=1, keepdims=True)
    lse1 = jnp.log(jnp.sum(jnp.exp(fs - m1), axis=1, keepdims=True)) + m1
    fsn = fs - lse1

    st = st_ref[...]
    k2 = st[:, 1:2] > st[:, 0:1]
    kb = k2[:, :, None]
    nlse = jnp.swapaxes(nlse_ref[...], 1, 2)
    ss_ref[1:N_FEATURES] = jnp.where(kb, nlse, fsn)

    kf = jnp.concatenate(
        [jnp.zeros((1, 16), jnp.int32),
         jnp.broadcast_to(k2.astype(jnp.int32), (N_FEATURES - 1, 16))], axis=0)
    kf_ref[...] = kf


def _small_prep(class_logits, feat0, feat_self, struct, nlse):
    return pl.pallas_call(
        _small_prep_body,
        out_shape=(
            jax.ShapeDtypeStruct((N_FEATURES, N_UNIQUE, N_CLASSES), jnp.float32),
            jax.ShapeDtypeStruct((N_FEATURES, 16), jnp.int32),
        ),
    )(class_logits.reshape(1, N_CLASSES), feat0, feat_self, struct, nlse)


def _accum(acc, buf, slot):
    def body(r, carry):
        for k4 in range(N_CLASSES // 16):
            sl = pl.ds(k4 * 16, 16)
            plsc.addupdate(acc.at[r, sl], buf[slot, r, sl])
        return carry
    lax.fori_loop(0, CHUNK, body, 0, unroll=4)


@functools.cache
def _build_sc_gather():
  mesh = plsc.VectorSubcoreMesh(
      core_axis_name="c", subcore_axis_name="s", num_cores=NC, num_subcores=NS)

  @functools.partial(
      pl.kernel,
      out_type=jax.ShapeDtypeStruct((BATCH, N_CLASSES), jnp.float32),
      mesh=mesh,
      scratch_types=[
          pltpu.VMEM((N_FEATURES, CHUNK), jnp.int32),
          pltpu.VMEM((N_FEATURES, 16), jnp.int32),
          pltpu.VMEM((NSLOT, NIDX, 128), jnp.int32),
          pltpu.VMEM((NSLOT, NIDX, 128), jnp.int32),
          pltpu.VMEM((NSLOT, CHUNK, N_CLASSES), jnp.float32),
          pltpu.VMEM((NSLOT, CHUNK, N_CLASSES), jnp.float32),
          pltpu.VMEM((CHUNK, N_CLASSES), jnp.float32),
      ] + [pltpu.SemaphoreType.DMA] * (2 * NSLOT),
      compiler_params=pltpu.CompilerParams(use_tc_tiling_on_sc=False),
  )
  def _sc_gather(xT_hbm, ss_hbm, pair_hbm, kf_hbm, out_hbm,
                 xv, kv, idxs, idxp, buf_s, buf_p, acc, *sems):
      wid = lax.axis_index("s") * NC + lax.axis_index("c")
      pltpu.sync_copy(kf_hbm, kv)
      sem_s = sems[:NSLOT]
      sem_p = sems[NSLOT:]

      def chunk_body(ci, carry):
          cbase = wid * SAMP_PER_W + ci * CHUNK
          pltpu.sync_copy(xT_hbm.at[:, pl.ds(cbase, CHUNK)], xv)

          def compute_idx(i, slot):
              kvec = kv[i, :]
              for c in range(CHUNK // 16):
                  j, o = divmod(c * 16, 128)
                  sl = pl.ds(c * 16, 16)
                  if i == 0:
                      idxs[slot, j, pl.ds(o, 16)] = xv[0, sl]
                  else:
                      xc = xv[i, sl]
                      xp = xv[i - 1, sl]
                      idxs[slot, j, pl.ds(o, 16)] = (
                          jnp.where(kvec > 0, xp, xc) + i * N_UNIQUE)
                      idxp[slot, j, pl.ds(o, 16)] = (
                          (i - 1) * PAIR_ROWS + xc * N_UNIQUE + xp)

          def fire(i, slot):
              dst = acc if i == 0 else buf_s.at[slot]
              hs, hp = [], []
              for j in range(NIDX):
                  hs.append(pltpu.make_async_copy(
                      ss_hbm.at[idxs.at[slot, j]],
                      dst.at[pl.ds(j * 128, 128)], sem_s[slot]))
                  hs[-1].start()
              if i > 0:
                  for j in range(NIDX):
                      hp.append(pltpu.make_async_copy(
                          pair_hbm.at[idxp.at[slot, j]],
                          buf_p.at[slot, pl.ds(j * 128, 128)], sem_p[slot]))
                      pl.when(kv[i, :][0] > 0)(hp[-1].start)
              return hs, hp

          def drain(i, slot, handles):
              hs, hp = handles
              for hj in hs:
                  hj.wait()
              if i > 0:
                  _accum(acc, buf_s, slot)

                  @pl.when(kv[i, :][0] > 0)
                  def _():
                      for hj in hp:
                          hj.wait()
                      _accum(acc, buf_p, slot)

          lead = NSLOT - 1
          h = {}
          for i in range(lead):
              compute_idx(i, i % NSLOT)
              h[i] = fire(i, i % NSLOT)
          for i in range(lead, N_FEATURES):
              compute_idx(i, i % NSLOT)
              h[i] = fire(i, i % NSLOT)
              d = i - lead
              drain(d, d % NSLOT, h[d])
          for d in range(N_FEATURES - lead, N_FEATURES):
              drain(d, d % NSLOT, h[d])

          pltpu.sync_copy(acc, out_hbm.at[pl.ds(cbase, CHUNK), :])
          return carry

      lax.fori_loop(0, NCHUNKS, chunk_body, 0)

  return _sc_gather


def kernel(x, training, class_logits, feat0, feat_self, feat_pair, struct):
    del training
    xT = x.astype(jnp.int32).T
    pair_flat = feat_pair.reshape(-1, N_CLASSES)
    nlse = _pair_lse(feat_pair)
    ss, kf = _small_prep(class_logits, feat0, feat_self, struct, nlse)
    ss_flat = ss.reshape(-1, N_CLASSES)
    return _build_sc_gather()(xT, ss_flat, pair_flat, kf)


# speedup vs baseline: 8.9254x; 1.3401x over previous
import functools

import jax
import jax.numpy as jnp
from jax import lax
from jax.experimental import pallas as pl
from jax.experimental.pallas import tpu as pltpu
from jax.experimental.pallas import tpu_sc as plsc

N_FEATURES = 26
N_UNIQUE = 100
N_CLASSES = 64
BATCH = 16384

NC = 2
NS = 16
NW = NC * NS
SAMP_PER_W = BATCH // NW
CHUNK = 128
NCHUNKS = SAMP_PER_W // CHUNK
NIDX = CHUNK // 128
NSLOT = 4
PAIR_ROWS = N_UNIQUE * N_UNIQUE


def _pair_lse_body(fpT_ref, out_ref, pack_ref):
    xb = fpT_ref[0]
    m = jnp.max(xb, axis=0)
    s = jnp.sum(jnp.exp(xb - m[None]), axis=0)
    out_ref[0] = -(jnp.log(s) + m)
    perm = jnp.concatenate(
        [jnp.arange(0, N_UNIQUE, 2), jnp.arange(1, N_UNIQUE, 2)])
    permb = jnp.broadcast_to(perm.reshape(1, 1, N_UNIQUE),
                             (N_UNIQUE, N_CLASSES, N_UNIQUE))
    xg = jnp.take_along_axis(xb, permb, axis=2)
    tt = jnp.swapaxes(xg, 1, 2)
    cat = jnp.concatenate(
        [tt[:, :N_UNIQUE // 2, :], tt[:, N_UNIQUE // 2:, :]], axis=2)
    pack_ref[0] = cat.reshape(PAIR_ROWS // 2, 2 * N_CLASSES)


def _pair_lse(feat_pair):
    fpT = jnp.transpose(feat_pair, (0, 1, 3, 2))
    out, pack = pl.pallas_call(
        _pair_lse_body,
        grid=(N_FEATURES - 1,),
        in_specs=[pl.BlockSpec((1, N_UNIQUE, N_CLASSES, N_UNIQUE),
                               lambda i: (i, 0, 0, 0))],
        out_specs=(
            pl.BlockSpec((1, N_CLASSES, N_UNIQUE), lambda i: (i, 0, 0)),
            pl.BlockSpec((1, PAIR_ROWS // 2, 2 * N_CLASSES),
                         lambda i: (i, 0, 0)),
        ),
        out_shape=(
            jax.ShapeDtypeStruct((N_FEATURES - 1, N_CLASSES, N_UNIQUE),
                                 jnp.float32),
            jax.ShapeDtypeStruct(
                (N_FEATURES - 1, PAIR_ROWS // 2, 2 * N_CLASSES), jnp.float32),
        ),
    )(fpT)
    return out, pack


def _small_prep_body(cl_ref, f0_ref, fs_ref, st_ref, nlse_ref, ss_ref, kf_ref):
    cl = cl_ref[...]
    m = jnp.max(cl, axis=1, keepdims=True)
    base = cl - (jnp.log(jnp.sum(jnp.exp(cl - m), axis=1, keepdims=True)) + m)

    f0 = f0_ref[...]
    m0 = jnp.max(f0, axis=0, keepdims=True)
    lse0 = jnp.log(jnp.sum(jnp.exp(f0 - m0), axis=0, keepdims=True)) + m0
    ss_ref[0] = f0 - lse0 + base

    fs = fs_ref[...]
    m1 = jnp.max(fs, axis=1, keepdims=True)
    lse1 = jnp.log(jnp.sum(jnp.exp(fs - m1), axis=1, keepdims=True)) + m1
    fsn = fs - lse1

    st = st_ref[...]
    k2 = st[:, 1:2] > st[:, 0:1]
    kb = k2[:, :, None]
    nlse = jnp.swapaxes(nlse_ref[...], 1, 2)
    ss_ref[1:N_FEATURES] = jnp.where(kb, nlse, fsn)

    kf = jnp.concatenate(
        [jnp.zeros((1, 16), jnp.int32),
         jnp.broadcast_to(k2.astype(jnp.int32), (N_FEATURES - 1, 16))], axis=0)
    kf_ref[...] = kf


def _small_prep(class_logits, feat0, feat_self, struct, nlse):
    return pl.pallas_call(
        _small_prep_body,
        out_shape=(
            jax.ShapeDtypeStruct((N_FEATURES, N_UNIQUE, N_CLASSES), jnp.float32),
            jax.ShapeDtypeStruct((N_FEATURES, 16), jnp.int32),
        ),
    )(class_logits.reshape(1, N_CLASSES), feat0, feat_self, struct, nlse)


def _accum(acc, buf, slot):
    def body(r, carry):
        for k4 in range(N_CLASSES // 16):
            sl = pl.ds(k4 * 16, 16)
            plsc.addupdate(acc.at[r, sl], buf[slot, r, sl])
        return carry
    lax.fori_loop(0, CHUNK, body, 0, unroll=4)


@functools.cache
def _build_sc_gather():
  mesh = plsc.VectorSubcoreMesh(
      core_axis_name="c", subcore_axis_name="s", num_cores=NC, num_subcores=NS)

  @functools.partial(
      pl.kernel,
      out_type=jax.ShapeDtypeStruct((BATCH, N_CLASSES), jnp.float32),
      mesh=mesh,
      scratch_types=[
          pltpu.VMEM((N_FEATURES, CHUNK), jnp.int32),
          pltpu.VMEM((N_FEATURES, 16), jnp.int32),
          pltpu.VMEM((NSLOT, NIDX, 128), jnp.int32),
          pltpu.VMEM((NSLOT, NIDX, 128), jnp.int32),
          pltpu.VMEM((NSLOT, CHUNK, N_CLASSES), jnp.float32),
          pltpu.VMEM((NSLOT, CHUNK, N_CLASSES), jnp.float32),
          pltpu.VMEM((CHUNK, N_CLASSES), jnp.float32),
      ] + [pltpu.SemaphoreType.DMA] * (2 * NSLOT),
      compiler_params=pltpu.CompilerParams(use_tc_tiling_on_sc=False),
  )
  def _sc_gather(xT_hbm, ss_hbm, pair_hbm, kf_hbm, out_hbm,
                 xv, kv, idxs, idxp, buf_s, buf_p, acc, *sems):
      wid = lax.axis_index("s") * NC + lax.axis_index("c")
      pltpu.sync_copy(kf_hbm, kv)
      sem_s = sems[:NSLOT]
      sem_p = sems[NSLOT:]

      def chunk_body(ci, carry):
          cbase = wid * SAMP_PER_W + ci * CHUNK
          pltpu.sync_copy(xT_hbm.at[:, pl.ds(cbase, CHUNK)], xv)

          def compute_idx(i, slot):
              kvec = kv[i, :]
              for c in range(CHUNK // 16):
                  j, o = divmod(c * 16, 128)
                  sl = pl.ds(c * 16, 16)
                  if i == 0:
                      idxs[slot, j, pl.ds(o, 16)] = xv[0, sl]
                  else:
                      xc = xv[i, sl]
                      xp = xv[i - 1, sl]
                      idxs[slot, j, pl.ds(o, 16)] = (
                          jnp.where(kvec > 0, xp, xc) + i * N_UNIQUE)
                      idxp[slot, j, pl.ds(o, 16)] = (
                          (i - 1) * PAIR_ROWS + xc * N_UNIQUE + xp)

          def fire(i, slot):
              dst = acc if i == 0 else buf_s.at[slot]
              hs, hp = [], []
              for j in range(NIDX):
                  hs.append(pltpu.make_async_copy(
                      ss_hbm.at[idxs.at[slot, j]],
                      dst.at[pl.ds(j * 128, 128)], sem_s[slot]))
                  hs[-1].start()
              if i > 0:
                  for j in range(NIDX):
                      hp.append(pltpu.make_async_copy(
                          pair_hbm.at[idxp.at[slot, j]],
                          buf_p.at[slot, pl.ds(j * 128, 128)], sem_p[slot]))
                      pl.when(kv[i, :][0] > 0)(hp[-1].start)
              return hs, hp

          def drain(i, slot, handles):
              hs, hp = handles
              for hj in hs:
                  hj.wait()
              if i > 0:
                  _accum(acc, buf_s, slot)

                  @pl.when(kv[i, :][0] > 0)
                  def _():
                      for hj in hp:
                          hj.wait()
                      _accum(acc, buf_p, slot)

          lead = NSLOT - 1
          h = {}
          for i in range(lead):
              compute_idx(i, i % NSLOT)
              h[i] = fire(i, i % NSLOT)
          for i in range(lead, N_FEATURES):
              compute_idx(i, i % NSLOT)
              h[i] = fire(i, i % NSLOT)
              d = i - lead
              drain(d, d % NSLOT, h[d])
          for d in range(N_FEATURES - lead, N_FEATURES):
              drain(d, d % NSLOT, h[d])

          pltpu.sync_copy(acc, out_hbm.at[pl.ds(cbase, CHUNK), :])
          return carry

      lax.fori_loop(0, NCHUNKS, chunk_body, 0)

  return _sc_gather


def kernel(x, training, class_logits, feat0, feat_self, feat_pair, struct):
    del training
    xT = x.astype(jnp.int32).T
    nlse, pack = _pair_lse(feat_pair)
    ss, kf = _small_prep(class_logits, feat0, feat_self, struct, nlse)
    ss_flat = ss.reshape(-1, N_CLASSES)
    pair_flat = pack.reshape(-1, N_CLASSES)
    return _build_sc_gather()(xT, ss_flat, pair_flat, kf)


# speedup vs baseline: 9.6479x; 1.0809x over previous
import functools

import jax
import jax.numpy as jnp
from jax import lax
from jax.experimental import pallas as pl
from jax.experimental.pallas import tpu as pltpu
from jax.experimental.pallas import tpu_sc as plsc

N_FEATURES = 26
N_UNIQUE = 100
N_CLASSES = 64
BATCH = 16384

NC = 2
NS = 16
NW = NC * NS
SAMP_PER_W = BATCH // NW
CHUNK = 128
NCHUNKS = SAMP_PER_W // CHUNK
NIDX = CHUNK // 128
NSLOT = 4
PAIR_ROWS = N_UNIQUE * N_UNIQUE


def _pair_lse_body(fpT_ref, out_ref, pack_ref):
    xb = fpT_ref[0]
    m = jnp.max(xb, axis=0)
    s = jnp.sum(jnp.exp(xb - m[None]), axis=0)
    out_ref[0] = -(jnp.log(s) + m)
    tt = jnp.swapaxes(xb, 1, 2)
    cat = jnp.concatenate(
        [tt[:, :N_UNIQUE // 2, :], tt[:, N_UNIQUE // 2:, :]], axis=2)
    pack_ref[0] = cat.reshape(PAIR_ROWS // 2, 2 * N_CLASSES)


def _pair_lse(feat_pair):
    fpT = jnp.transpose(feat_pair, (0, 1, 3, 2))
    out, pack = pl.pallas_call(
        _pair_lse_body,
        grid=(N_FEATURES - 1,),
        in_specs=[pl.BlockSpec((1, N_UNIQUE, N_CLASSES, N_UNIQUE),
                               lambda i: (i, 0, 0, 0))],
        out_specs=(
            pl.BlockSpec((1, N_CLASSES, N_UNIQUE), lambda i: (i, 0, 0)),
            pl.BlockSpec((1, PAIR_ROWS // 2, 2 * N_CLASSES),
                         lambda i: (i, 0, 0)),
        ),
        out_shape=(
            jax.ShapeDtypeStruct((N_FEATURES - 1, N_CLASSES, N_UNIQUE),
                                 jnp.float32),
            jax.ShapeDtypeStruct(
                (N_FEATURES - 1, PAIR_ROWS // 2, 2 * N_CLASSES), jnp.float32),
        ),
    )(fpT)
    return out, pack


def _small_prep_body(cl_ref, f0_ref, fs_ref, st_ref, nlse_ref, ss_ref, kf_ref):
    cl = cl_ref[...]
    m = jnp.max(cl, axis=1, keepdims=True)
    base = cl - (jnp.log(jnp.sum(jnp.exp(cl - m), axis=1, keepdims=True)) + m)

    f0 = f0_ref[...]
    m0 = jnp.max(f0, axis=0, keepdims=True)
    lse0 = jnp.log(jnp.sum(jnp.exp(f0 - m0), axis=0, keepdims=True)) + m0
    ss_ref[0] = f0 - lse0 + base

    fs = fs_ref[...]
    m1 = jnp.max(fs, axis=1, keepdims=True)
    lse1 = jnp.log(jnp.sum(jnp.exp(fs - m1), axis=1, keepdims=True)) + m1
    fsn = fs - lse1

    st = st_ref[...]
    k2 = st[:, 1:2] > st[:, 0:1]
    kb = k2[:, :, None]
    nlse = jnp.swapaxes(nlse_ref[...], 1, 2)
    ss_ref[1:N_FEATURES] = jnp.where(kb, nlse, fsn)

    kf = jnp.concatenate(
        [jnp.zeros((1, 16), jnp.int32),
         jnp.broadcast_to(k2.astype(jnp.int32), (N_FEATURES - 1, 16))], axis=0)
    kf_ref[...] = kf


def _small_prep(class_logits, feat0, feat_self, struct, nlse):
    return pl.pallas_call(
        _small_prep_body,
        out_shape=(
            jax.ShapeDtypeStruct((N_FEATURES, N_UNIQUE, N_CLASSES), jnp.float32),
            jax.ShapeDtypeStruct((N_FEATURES, 16), jnp.int32),
        ),
    )(class_logits.reshape(1, N_CLASSES), feat0, feat_self, struct, nlse)


def _accum(acc, buf, slot):
    def body(r, carry):
        for k4 in range(N_CLASSES // 16):
            sl = pl.ds(k4 * 16, 16)
            plsc.addupdate(acc.at[r, sl], buf[slot, r, sl])
        return carry
    lax.fori_loop(0, CHUNK, body, 0, unroll=4)


@functools.cache
def _build_sc_gather():
  mesh = plsc.VectorSubcoreMesh(
      core_axis_name="c", subcore_axis_name="s", num_cores=NC, num_subcores=NS)

  @functools.partial(
      pl.kernel,
      out_type=jax.ShapeDtypeStruct((BATCH, N_CLASSES), jnp.float32),
      mesh=mesh,
      scratch_types=[
          pltpu.VMEM((N_FEATURES, CHUNK), jnp.int32),
          pltpu.VMEM((N_FEATURES, 16), jnp.int32),
          pltpu.VMEM((NSLOT, NIDX, 128), jnp.int32),
          pltpu.VMEM((NSLOT, NIDX, 128), jnp.int32),
          pltpu.VMEM((NSLOT, CHUNK, N_CLASSES), jnp.float32),
          pltpu.VMEM((NSLOT, CHUNK, N_CLASSES), jnp.float32),
          pltpu.VMEM((CHUNK, N_CLASSES), jnp.float32),
      ] + [pltpu.SemaphoreType.DMA] * (2 * NSLOT),
      compiler_params=pltpu.CompilerParams(use_tc_tiling_on_sc=False),
  )
  def _sc_gather(xT_hbm, ss_hbm, pair_hbm, kf_hbm, out_hbm,
                 xv, kv, idxs, idxp, buf_s, buf_p, acc, *sems):
      wid = lax.axis_index("s") * NC + lax.axis_index("c")
      pltpu.sync_copy(kf_hbm, kv)
      sem_s = sems[:NSLOT]
      sem_p = sems[NSLOT:]

      def chunk_body(ci, carry):
          cbase = wid * SAMP_PER_W + ci * CHUNK
          pltpu.sync_copy(xT_hbm.at[:, pl.ds(cbase, CHUNK)], xv)

          def compute_idx(i, slot):
              kvec = kv[i, :]
              for c in range(CHUNK // 16):
                  j, o = divmod(c * 16, 128)
                  sl = pl.ds(c * 16, 16)
                  if i == 0:
                      idxs[slot, j, pl.ds(o, 16)] = xv[0, sl]
                  else:
                      xc = xv[i, sl]
                      xp = xv[i - 1, sl]
                      idxs[slot, j, pl.ds(o, 16)] = (
                          jnp.where(kvec > 0, xp, xc) + i * N_UNIQUE)
                      q = jnp.where(xp >= N_UNIQUE // 2, 1, 0)
                      idxp[slot, j, pl.ds(o, 16)] = (
                          (i - 1) * PAIR_ROWS + xc * N_UNIQUE
                          + 2 * xp - (N_UNIQUE - 1) * q)

          def fire(i, slot):
              dst = acc if i == 0 else buf_s.at[slot]
              hs, hp = [], []
              for j in range(NIDX):
                  hs.append(pltpu.make_async_copy(
                      ss_hbm.at[idxs.at[slot, j]],
                      dst.at[pl.ds(j * 128, 128)], sem_s[slot]))
                  hs[-1].start()
              if i > 0:
                  for j in range(NIDX):
                      hp.append(pltpu.make_async_copy(
                          pair_hbm.at[idxp.at[slot, j]],
                          buf_p.at[slot, pl.ds(j * 128, 128)], sem_p[slot]))
                      pl.when(kv[i, :][0] > 0)(hp[-1].start)
              return hs, hp

          def drain(i, slot, handles):
              hs, hp = handles
              for hj in hs:
                  hj.wait()
              if i > 0:
                  _accum(acc, buf_s, slot)

                  @pl.when(kv[i, :][0] > 0)
                  def _():
                      for hj in hp:
                          hj.wait()
                      _accum(acc, buf_p, slot)

          lead = NSLOT - 1
          h = {}
          for i in range(lead):
              compute_idx(i, i % NSLOT)
              h[i] = fire(i, i % NSLOT)
          for i in range(lead, N_FEATURES):
              compute_idx(i, i % NSLOT)
              h[i] = fire(i, i % NSLOT)
              d = i - lead
              drain(d, d % NSLOT, h[d])
          for d in range(N_FEATURES - lead, N_FEATURES):
              drain(d, d % NSLOT, h[d])

          pltpu.sync_copy(acc, out_hbm.at[pl.ds(cbase, CHUNK), :])
          return carry

      lax.fori_loop(0, NCHUNKS, chunk_body, 0)

  return _sc_gather


def kernel(x, training, class_logits, feat0, feat_self, feat_pair, struct):
    del training
    xT = x.astype(jnp.int32).T
    nlse, pack = _pair_lse(feat_pair)
    ss, kf = _small_prep(class_logits, feat0, feat_self, struct, nlse)
    ss_flat = ss.reshape(-1, N_CLASSES)
    pair_flat = pack.reshape(-1, N_CLASSES)
    return _build_sc_gather()(xT, ss_flat, pair_flat, kf)
